# Initial kernel scaffold; baseline (speedup 1.0000x reference)
import jax
import jax.numpy as jnp
from jax.experimental import pallas as pl


def kernel(x, edge_index, edge_attr, info_batch, W1, b1, c1_w, c1_b, W2, b2, c2_w, c2_b, W3, b3, c3_w, c3_b):
    raise NotImplementedError("write your pallas kernel here")


# speedup vs baseline: 5.9089x; 5.9089x over previous
import functools

import jax
import jax.numpy as jnp
from jax import lax
from jax.experimental import pallas as pl
from jax.experimental.pallas import tpu as pltpu
from jax.experimental.pallas import tpu_sc as plsc

N = 10000
E_RAW = 320000
EP = 330240
NC = 2
NS = 16
L = 16
EPT = EP // (NC * NS)
NG = EPT // L
RPT = 624
ZR = 16
D_IN = 128
D1 = 256
D2 = 256
D3 = 128
G = 64
R = 2000

_mesh = plsc.VectorSubcoreMesh(core_axis_name="c", subcore_axis_name="s")
_sc_params = pltpu.CompilerParams(needs_layout_passes=False)


@functools.partial(
    pl.kernel, mesh=_mesh, compiler_params=_sc_params,
    out_type=jax.ShapeDtypeStruct((NC * NS * N,), jnp.float32),
    scratch_types=[
        pltpu.VMEM((EPT,), jnp.int32),
        pltpu.VMEM((EPT,), jnp.float32),
        pltpu.VMEM((N,), jnp.float32),
    ],
)
def _sc_deg(dst_hbm, ew_hbm, out_hbm, dst_v, ew_v, deg_v):
    cid = lax.axis_index("c")
    sid = lax.axis_index("s")
    wid = cid * NS + sid
    base = pl.multiple_of(wid * EPT, 8)
    pltpu.sync_copy(dst_hbm.at[pl.ds(base, EPT)], dst_v)
    pltpu.sync_copy(ew_hbm.at[pl.ds(base, EPT)], ew_v)
    z16 = jnp.zeros((L,), jnp.float32)

    def zero_body(i, c):
        deg_v[pl.ds(i * L, L)] = z16
        return c

    lax.fori_loop(0, N // L, zero_body, 0)

    def body(g, c):
        d16 = dst_v[pl.ds(g * L, L)]
        e16 = ew_v[pl.ds(g * L, L)]
        plsc.addupdate_scatter(deg_v, [d16], e16)
        return c

    lax.fori_loop(0, NG, body, 0)
    pltpu.sync_copy(deg_v, out_hbm.at[pl.ds(pl.multiple_of(wid * N, 8), N)])


@functools.partial(
    pl.kernel, mesh=_mesh, compiler_params=_sc_params,
    out_type=jax.ShapeDtypeStruct((EP,), jnp.float32),
    scratch_types=[
        pltpu.VMEM((EPT,), jnp.int32),
        pltpu.VMEM((EPT,), jnp.int32),
        pltpu.VMEM((EPT,), jnp.float32),
        pltpu.VMEM((EPT,), jnp.float32),
        pltpu.VMEM((N,), jnp.float32),
    ],
)
def _sc_norm(src_hbm, dst_hbm, ew_hbm, dis_hbm, out_hbm,
             src_v, dst_v, ew_v, norm_v, dis_v):
    cid = lax.axis_index("c")
    sid = lax.axis_index("s")
    wid = cid * NS + sid
    base = pl.multiple_of(wid * EPT, 8)
    pltpu.sync_copy(src_hbm.at[pl.ds(base, EPT)], src_v)
    pltpu.sync_copy(dst_hbm.at[pl.ds(base, EPT)], dst_v)
    pltpu.sync_copy(ew_hbm.at[pl.ds(base, EPT)], ew_v)
    pltpu.sync_copy(dis_hbm, dis_v)

    def body(g, c):
        s16 = src_v[pl.ds(g * L, L)]
        d16 = dst_v[pl.ds(g * L, L)]
        e16 = ew_v[pl.ds(g * L, L)]
        a = plsc.load_gather(dis_v, [s16])
        b = plsc.load_gather(dis_v, [d16])
        norm_v[pl.ds(g * L, L)] = a * e16 * b
        return c

    lax.fori_loop(0, NG, body, 0)
    pltpu.sync_copy(norm_v, out_hbm.at[pl.ds(base, EPT)])


@functools.partial(
    pl.kernel, mesh=_mesh, compiler_params=_sc_params,
    out_type=jax.ShapeDtypeStruct((NC * N, D_IN), jnp.float32),
    scratch_types=[
        pltpu.VMEM((EPT,), jnp.int32),
        pltpu.VMEM((EPT,), jnp.int32),
        pltpu.VMEM((EPT,), jnp.float32),
        pltpu.VMEM((L, D_IN), jnp.float32),
        pltpu.VMEM((L,), jnp.int32),
        pltpu.VMEM((L,), jnp.int32),
        pltpu.VMEM((ZR, D_IN), jnp.float32),
        pltpu.VMEM_SHARED((N, D_IN), jnp.float32),
        pltpu.SemaphoreType.DMA,
    ],
)
def _sc_agg(feat_hbm, src_hbm, dst_hbm, norm_hbm, out_hbm,
            src_v, dst_v, norm_v, rows_v, sidx, didx, zbuf, acc, sem):
    cid = lax.axis_index("c")
    sid = lax.axis_index("s")
    wid = cid * NS + sid
    base = pl.multiple_of(wid * EPT, 8)
    pltpu.sync_copy(src_hbm.at[pl.ds(base, EPT)], src_v)
    pltpu.sync_copy(dst_hbm.at[pl.ds(base, EPT)], dst_v)
    pltpu.sync_copy(norm_hbm.at[pl.ds(base, EPT)], norm_v)

    z16 = jnp.zeros((L,), jnp.float32)

    def zb_body(i, c):
        for k in range(D_IN // L):
            zbuf[i, pl.ds(k * L, L)] = z16
        return c

    lax.fori_loop(0, ZR, zb_body, 0)
    off = pl.multiple_of(sid * RPT, 8)

    def zc_body(j, c):
        pltpu.sync_copy(zbuf, acc.at[pl.ds(pl.multiple_of(off + j * ZR, 8),
                                           ZR)])
        return c

    lax.fori_loop(0, RPT // ZR, zc_body, 0)

    @pl.when(sid == NS - 1)
    def _():
        pltpu.sync_copy(zbuf, acc.at[pl.ds(NS * RPT, ZR)])

    plsc.subcore_barrier()

    def body(g, c):
        s16 = src_v[pl.ds(g * L, L)]
        d16 = dst_v[pl.ds(g * L, L)]
        sidx[...] = s16
        didx[...] = d16
        pltpu.async_copy(feat_hbm.at[sidx], rows_v, sem).wait()
        n16 = norm_v[pl.ds(g * L, L)]
        for r in range(L):
            nv = n16[r]
            for k in range(D_IN // L):
                rows_v[r, pl.ds(k * L, L)] = rows_v[r, pl.ds(k * L, L)] * nv
        pltpu.sync_copy(rows_v, acc.at[didx], add=True)
        return c

    lax.fori_loop(0, NG, body, 0)
    plsc.subcore_barrier()

    @pl.when(sid < NS - 1)
    def _():
        pltpu.sync_copy(
            acc.at[pl.ds(off, RPT)],
            out_hbm.at[pl.ds(pl.multiple_of(cid * N + off, 8), RPT)])

    @pl.when(sid == NS - 1)
    def _():
        last = NS - 1
        rem = N - last * RPT
        pltpu.sync_copy(
            acc.at[pl.ds(last * RPT, rem)],
            out_hbm.at[pl.ds(pl.multiple_of(cid * N + last * RPT, 8), rem)])


def _tc_dis_body(degp_ref, out_ref):
    deg = jnp.sum(degp_ref[...], axis=0, keepdims=True)
    safe = jnp.where(deg > 0, deg, 1.0)
    out_ref[...] = jnp.where(deg > 0, lax.rsqrt(safe), 0.0)


def _tc_dis(degp):
    return pl.pallas_call(
        _tc_dis_body,
        out_shape=jax.ShapeDtypeStruct((1, N), jnp.float32),
    )(degp)


def _conv_mat(c_ref, d):
    w0 = c_ref[0, 0]
    w1 = c_ref[0, 1]
    w2 = c_ref[0, 2]
    ii = lax.broadcasted_iota(jnp.int32, (d, d), 0)
    jj = lax.broadcasted_iota(jnp.int32, (d, d), 1)
    delta = jj - ii
    return (jnp.where(delta == 1, w0, 0.0)
            + jnp.where(delta == 0, w1, 0.0)
            + jnp.where(delta == -1, w2, 0.0))


def _tc_l1_body(p_ref, w_ref, b_ref, c_ref, cb_ref, outa_ref, outb_ref):
    a = p_ref[0] + p_ref[1]
    t = jnp.maximum(jnp.dot(a, w_ref[...],
                            preferred_element_type=jnp.float32)
                    + b_ref[...], 0.0)
    cm = _conv_mat(c_ref, D1)
    y = jnp.dot(t, cm, preferred_element_type=jnp.float32) + cb_ref[0, 0]
    h = jnp.maximum(y, 0.0)
    outa_ref[...] = h[:, :D_IN]
    outb_ref[...] = h[:, D_IN:]


def _tc_l1(p, w1, b1, c1, c1b):
    return pl.pallas_call(
        _tc_l1_body,
        grid=(N // R,),
        in_specs=[
            pl.BlockSpec((NC, R, D_IN), lambda i: (0, i, 0)),
            pl.BlockSpec((D_IN, D1), lambda i: (0, 0)),
            pl.BlockSpec((1, D1), lambda i: (0, 0)),
            pl.BlockSpec(memory_space=pltpu.SMEM),
            pl.BlockSpec(memory_space=pltpu.SMEM),
        ],
        out_specs=[
            pl.BlockSpec((R, D_IN), lambda i: (i, 0)),
            pl.BlockSpec((R, D_IN), lambda i: (i, 0)),
        ],
        out_shape=[
            jax.ShapeDtypeStruct((N, D_IN), jnp.float32),
            jax.ShapeDtypeStruct((N, D_IN), jnp.float32),
        ],
    )(p, w1, b1, c1, c1b)


def _tc_l2_body(pa_ref, pb_ref, w2_ref, b2_ref, c_ref, cb_ref, w3_ref,
                out_ref):
    a = jnp.concatenate([pa_ref[0] + pa_ref[1], pb_ref[0] + pb_ref[1]],
                        axis=1)
    t = jnp.maximum(jnp.dot(a, w2_ref[...],
                            preferred_element_type=jnp.float32)
                    + b2_ref[...], 0.0)
    cm = _conv_mat(c_ref, D2)
    y = jnp.dot(t, cm, preferred_element_type=jnp.float32) + cb_ref[0, 0]
    h = jnp.maximum(y, 0.0)
    out_ref[...] = jnp.dot(h, w3_ref[...], preferred_element_type=jnp.float32)


def _tc_l2(pa, pb, w2, b2, c2, c2b, w3):
    return pl.pallas_call(
        _tc_l2_body,
        grid=(N // R,),
        in_specs=[
            pl.BlockSpec((NC, R, D_IN), lambda i: (0, i, 0)),
            pl.BlockSpec((NC, R, D_IN), lambda i: (0, i, 0)),
            pl.BlockSpec((D1, D2), lambda i: (0, 0)),
            pl.BlockSpec((1, D2), lambda i: (0, 0)),
            pl.BlockSpec(memory_space=pltpu.SMEM),
            pl.BlockSpec(memory_space=pltpu.SMEM),
            pl.BlockSpec((D2, D3), lambda i: (0, 0)),
        ],
        out_specs=pl.BlockSpec((R, D3), lambda i: (i, 0)),
        out_shape=jax.ShapeDtypeStruct((N, D3), jnp.float32),
    )(pa, pb, w2, b2, c2, c2b, w3)


def _tc_l3_body(p_ref, b3_ref, c_ref, cb_ref, ib_ref, out_ref,
                sums_ref, cnts_ref):
    i = pl.program_id(0)

    @pl.when(i == 0)
    def _():
        sums_ref[...] = jnp.zeros((G, D3), jnp.float32)
        cnts_ref[...] = jnp.zeros((G, D3), jnp.float32)

    a = p_ref[0] + p_ref[1]
    t = jnp.maximum(a + b3_ref[...], 0.0)
    cm = _conv_mat(c_ref, D3)
    y = jnp.dot(t, cm, preferred_element_type=jnp.float32) + cb_ref[0, 0]
    h = jnp.maximum(y, 0.0)
    ids = ib_ref[0]
    gids = lax.broadcasted_iota(jnp.int32, (G, R), 0)
    onehot = jnp.where(gids == ids, 1.0, 0.0)
    sums_ref[...] = sums_ref[...] + jnp.dot(
        onehot, h, preferred_element_type=jnp.float32)
    cnts_ref[...] = cnts_ref[...] + jnp.sum(onehot, axis=1, keepdims=True)

    @pl.when(i == N // R - 1)
    def _():
        out_ref[...] = sums_ref[...] / jnp.maximum(cnts_ref[...], 1.0)


def _tc_l3(p, b3, c3, c3b, ib):
    return pl.pallas_call(
        _tc_l3_body,
        grid=(N // R,),
        in_specs=[
            pl.BlockSpec((NC, R, D3), lambda i: (0, i, 0)),
            pl.BlockSpec((1, D3), lambda i: (0, 0)),
            pl.BlockSpec(memory_space=pltpu.SMEM),
            pl.BlockSpec(memory_space=pltpu.SMEM),
            pl.BlockSpec((1, 1, R), lambda i: (i, 0, 0)),
        ],
        out_specs=pl.BlockSpec((G, D3), lambda i: (0, 0)),
        out_shape=jax.ShapeDtypeStruct((G, D3), jnp.float32),
        scratch_shapes=[
            pltpu.VMEM((G, D3), jnp.float32),
            pltpu.VMEM((G, D3), jnp.float32),
        ],
    )(p, b3, c3, c3b, ib)


def kernel(x, edge_index, edge_attr, info_batch, W1, b1, c1_w, c1_b,
           W2, b2, c2_w, c2_b, W3, b3, c3_w, c3_b):
    loop = jnp.arange(N, dtype=edge_index.dtype)
    pad = EP - (E_RAW + N)
    src = jnp.concatenate([edge_index[0], loop,
                           jnp.zeros((pad,), edge_index.dtype)])
    dst = jnp.concatenate([edge_index[1], loop,
                           jnp.zeros((pad,), edge_index.dtype)])
    ew = jnp.concatenate([edge_attr, jnp.ones((N,), edge_attr.dtype),
                          jnp.zeros((pad,), edge_attr.dtype)])

    degp = _sc_deg(dst, ew).reshape(NC * NS, N)
    dis = _tc_dis(degp).reshape(N)
    norm = _sc_norm(src, dst, ew, dis)

    aggx = _sc_agg(x, src, dst, norm).reshape(NC, N, D_IN)
    h1a, h1b = _tc_l1(aggx, W1, b1.reshape(1, D1),
                      c1_w.reshape(1, 3), c1_b.reshape(1, 1))

    agga = _sc_agg(h1a, src, dst, norm).reshape(NC, N, D_IN)
    aggb = _sc_agg(h1b, src, dst, norm).reshape(NC, N, D_IN)
    g3 = _tc_l2(agga, aggb, W2, b2.reshape(1, D2),
                c2_w.reshape(1, 3), c2_b.reshape(1, 1), W3)

    aggg = _sc_agg(g3, src, dst, norm).reshape(NC, N, D3)
    ib = info_batch.reshape(N // R, 1, R)
    return _tc_l3(aggg, b3.reshape(1, D3), c3_w.reshape(1, 3),
                  c3_b.reshape(1, 1), ib)


# speedup vs baseline: 17.6524x; 2.9874x over previous
import functools

import jax
import jax.numpy as jnp
from jax import lax
from jax.experimental import pallas as pl
from jax.experimental.pallas import tpu as pltpu
from jax.experimental.pallas import tpu_sc as plsc

N = 10000
E_RAW = 320000
EP = 330240
NC = 2
NS = 16
L = 16
EPT = EP // (NC * NS)
NG = EPT // L
RPT = 624
ZR = 16
D_IN = 128
D1 = 256
D2 = 256
D3 = 128
G = 64
R = 2000

_mesh = plsc.VectorSubcoreMesh(core_axis_name="c", subcore_axis_name="s")
_sc_params = pltpu.CompilerParams(needs_layout_passes=False)


@functools.partial(
    pl.kernel, mesh=_mesh, compiler_params=_sc_params,
    out_type=jax.ShapeDtypeStruct((NC * NS * N,), jnp.float32),
    scratch_types=[
        pltpu.VMEM((EPT,), jnp.int32),
        pltpu.VMEM((EPT,), jnp.float32),
        pltpu.VMEM((N,), jnp.float32),
    ],
)
def _sc_deg(dst_hbm, ew_hbm, out_hbm, dst_v, ew_v, deg_v):
    cid = lax.axis_index("c")
    sid = lax.axis_index("s")
    wid = cid * NS + sid
    base = pl.multiple_of(wid * EPT, 8)
    pltpu.sync_copy(dst_hbm.at[pl.ds(base, EPT)], dst_v)
    pltpu.sync_copy(ew_hbm.at[pl.ds(base, EPT)], ew_v)
    z16 = jnp.zeros((L,), jnp.float32)

    def zero_body(i, c):
        deg_v[pl.ds(i * L, L)] = z16
        return c

    lax.fori_loop(0, N // L, zero_body, 0)

    def body(g, c):
        d16 = dst_v[pl.ds(g * L, L)]
        e16 = ew_v[pl.ds(g * L, L)]
        plsc.addupdate_scatter(deg_v, [d16], e16)
        return c

    lax.fori_loop(0, NG, body, 0)
    pltpu.sync_copy(deg_v, out_hbm.at[pl.ds(pl.multiple_of(wid * N, 8), N)])


@functools.partial(
    pl.kernel, mesh=_mesh, compiler_params=_sc_params,
    out_type=jax.ShapeDtypeStruct((EP,), jnp.float32),
    scratch_types=[
        pltpu.VMEM((EPT,), jnp.int32),
        pltpu.VMEM((EPT,), jnp.int32),
        pltpu.VMEM((EPT,), jnp.float32),
        pltpu.VMEM((EPT,), jnp.float32),
        pltpu.VMEM((N,), jnp.float32),
    ],
)
def _sc_norm(src_hbm, dst_hbm, ew_hbm, dis_hbm, out_hbm,
             src_v, dst_v, ew_v, norm_v, dis_v):
    cid = lax.axis_index("c")
    sid = lax.axis_index("s")
    wid = cid * NS + sid
    base = pl.multiple_of(wid * EPT, 8)
    pltpu.sync_copy(src_hbm.at[pl.ds(base, EPT)], src_v)
    pltpu.sync_copy(dst_hbm.at[pl.ds(base, EPT)], dst_v)
    pltpu.sync_copy(ew_hbm.at[pl.ds(base, EPT)], ew_v)
    pltpu.sync_copy(dis_hbm, dis_v)

    def body(g, c):
        s16 = src_v[pl.ds(g * L, L)]
        d16 = dst_v[pl.ds(g * L, L)]
        e16 = ew_v[pl.ds(g * L, L)]
        a = plsc.load_gather(dis_v, [s16])
        b = plsc.load_gather(dis_v, [d16])
        norm_v[pl.ds(g * L, L)] = a * e16 * b
        return c

    lax.fori_loop(0, NG, body, 0)
    pltpu.sync_copy(norm_v, out_hbm.at[pl.ds(base, EPT)])


GE = 48
NG2 = EPT // GE


@functools.partial(
    pl.kernel, mesh=_mesh, compiler_params=_sc_params,
    out_type=jax.ShapeDtypeStruct((NC * N, D_IN), jnp.float32),
    scratch_types=[
        pltpu.VMEM((EPT,), jnp.int32),
        pltpu.VMEM((EPT,), jnp.int32),
        pltpu.VMEM((EPT,), jnp.float32),
        [pltpu.VMEM((GE, D_IN), jnp.float32) for _ in range(2)],
        [pltpu.VMEM((GE,), jnp.int32) for _ in range(2)],
        [pltpu.VMEM((GE,), jnp.int32) for _ in range(2)],
        pltpu.VMEM((ZR, D_IN), jnp.float32),
        pltpu.VMEM_SHARED((N, D_IN), jnp.float32),
        [pltpu.SemaphoreType.DMA for _ in range(2)],
    ],
)
def _sc_agg(feat_hbm, src_hbm, dst_hbm, norm_hbm, out_hbm,
            src_v, dst_v, norm_v, rows_v, sidx, didx, zbuf, acc, sem):
    cid = lax.axis_index("c")
    sid = lax.axis_index("s")
    wid = cid * NS + sid
    base = pl.multiple_of(wid * EPT, 8)
    pltpu.sync_copy(src_hbm.at[pl.ds(base, EPT)], src_v)
    pltpu.sync_copy(dst_hbm.at[pl.ds(base, EPT)], dst_v)
    pltpu.sync_copy(norm_hbm.at[pl.ds(base, EPT)], norm_v)

    z16 = jnp.zeros((L,), jnp.float32)

    def zb_body(i, c):
        for k in range(D_IN // L):
            zbuf[i, pl.ds(k * L, L)] = z16
        return c

    lax.fori_loop(0, ZR, zb_body, 0)
    off = pl.multiple_of(sid * RPT, 8)

    def zc_body(j, c):
        pltpu.sync_copy(zbuf, acc.at[pl.ds(pl.multiple_of(off + j * ZR, 8),
                                           ZR)])
        return c

    lax.fori_loop(0, RPT // ZR, zc_body, 0)

    @pl.when(sid == NS - 1)
    def _():
        pltpu.sync_copy(zbuf, acc.at[pl.ds(NS * RPT, ZR)])

    plsc.subcore_barrier()

    def stage(g, b):
        for k in range(GE // L):
            sidx[b][pl.ds(k * L, L)] = src_v[pl.ds(g * GE + k * L, L)]
            didx[b][pl.ds(k * L, L)] = dst_v[pl.ds(g * GE + k * L, L)]
        pltpu.make_async_copy(feat_hbm.at[sidx[b]], rows_v[b], sem[b]).start()

    def process(g, b):
        pltpu.make_async_copy(feat_hbm.at[sidx[b]], rows_v[b], sem[b]).wait()
        for k in range(GE // L):
            n16 = norm_v[pl.ds(g * GE + k * L, L)]
            for r in range(L):
                nv = n16[r]
                row = k * L + r
                for q in range(D_IN // L):
                    rows_v[b][row, pl.ds(q * L, L)] = (
                        rows_v[b][row, pl.ds(q * L, L)] * nv)
        pltpu.sync_copy(rows_v[b], acc.at[didx[b]], add=True)

    stage(0, 0)
    stage(1, 1)

    def body(i, c):
        for b in range(2):
            g = i * 2 + b
            process(g, b)

            @pl.when(g + 2 < NG2)
            def _():
                stage(g + 2, b)

        return c

    lax.fori_loop(0, NG2 // 2, body, 0)
    process(NG2 - 1, 0)
    plsc.subcore_barrier()

    @pl.when(sid < NS - 1)
    def _():
        pltpu.sync_copy(
            acc.at[pl.ds(off, RPT)],
            out_hbm.at[pl.ds(pl.multiple_of(cid * N + off, 8), RPT)])

    @pl.when(sid == NS - 1)
    def _():
        last = NS - 1
        rem = N - last * RPT
        pltpu.sync_copy(
            acc.at[pl.ds(last * RPT, rem)],
            out_hbm.at[pl.ds(pl.multiple_of(cid * N + last * RPT, 8), rem)])


def _tc_dis_body(degp_ref, out_ref):
    deg = jnp.sum(degp_ref[...], axis=0, keepdims=True)
    safe = jnp.where(deg > 0, deg, 1.0)
    out_ref[...] = jnp.where(deg > 0, lax.rsqrt(safe), 0.0)


def _tc_dis(degp):
    return pl.pallas_call(
        _tc_dis_body,
        out_shape=jax.ShapeDtypeStruct((1, N), jnp.float32),
    )(degp)


def _conv_mat(c_ref, d):
    w0 = c_ref[0, 0]
    w1 = c_ref[0, 1]
    w2 = c_ref[0, 2]
    ii = lax.broadcasted_iota(jnp.int32, (d, d), 0)
    jj = lax.broadcasted_iota(jnp.int32, (d, d), 1)
    delta = jj - ii
    return (jnp.where(delta == 1, w0, 0.0)
            + jnp.where(delta == 0, w1, 0.0)
            + jnp.where(delta == -1, w2, 0.0))


def _tc_l1_body(p_ref, w_ref, b_ref, c_ref, cb_ref, outa_ref, outb_ref):
    a = p_ref[0] + p_ref[1]
    t = jnp.maximum(jnp.dot(a, w_ref[...],
                            preferred_element_type=jnp.float32)
                    + b_ref[...], 0.0)
    cm = _conv_mat(c_ref, D1)
    y = jnp.dot(t, cm, preferred_element_type=jnp.float32) + cb_ref[0, 0]
    h = jnp.maximum(y, 0.0)
    outa_ref[...] = h[:, :D_IN]
    outb_ref[...] = h[:, D_IN:]


def _tc_l1(p, w1, b1, c1, c1b):
    return pl.pallas_call(
        _tc_l1_body,
        grid=(N // R,),
        in_specs=[
            pl.BlockSpec((NC, R, D_IN), lambda i: (0, i, 0)),
            pl.BlockSpec((D_IN, D1), lambda i: (0, 0)),
            pl.BlockSpec((1, D1), lambda i: (0, 0)),
            pl.BlockSpec(memory_space=pltpu.SMEM),
            pl.BlockSpec(memory_space=pltpu.SMEM),
        ],
        out_specs=[
            pl.BlockSpec((R, D_IN), lambda i: (i, 0)),
            pl.BlockSpec((R, D_IN), lambda i: (i, 0)),
        ],
        out_shape=[
            jax.ShapeDtypeStruct((N, D_IN), jnp.float32),
            jax.ShapeDtypeStruct((N, D_IN), jnp.float32),
        ],
    )(p, w1, b1, c1, c1b)


def _tc_l2_body(pa_ref, pb_ref, w2_ref, b2_ref, c_ref, cb_ref, w3_ref,
                out_ref):
    a = jnp.concatenate([pa_ref[0] + pa_ref[1], pb_ref[0] + pb_ref[1]],
                        axis=1)
    t = jnp.maximum(jnp.dot(a, w2_ref[...],
                            preferred_element_type=jnp.float32)
                    + b2_ref[...], 0.0)
    cm = _conv_mat(c_ref, D2)
    y = jnp.dot(t, cm, preferred_element_type=jnp.float32) + cb_ref[0, 0]
    h = jnp.maximum(y, 0.0)
    out_ref[...] = jnp.dot(h, w3_ref[...], preferred_element_type=jnp.float32)


def _tc_l2(pa, pb, w2, b2, c2, c2b, w3):
    return pl.pallas_call(
        _tc_l2_body,
        grid=(N // R,),
        in_specs=[
            pl.BlockSpec((NC, R, D_IN), lambda i: (0, i, 0)),
            pl.BlockSpec((NC, R, D_IN), lambda i: (0, i, 0)),
            pl.BlockSpec((D1, D2), lambda i: (0, 0)),
            pl.BlockSpec((1, D2), lambda i: (0, 0)),
            pl.BlockSpec(memory_space=pltpu.SMEM),
            pl.BlockSpec(memory_space=pltpu.SMEM),
            pl.BlockSpec((D2, D3), lambda i: (0, 0)),
        ],
        out_specs=pl.BlockSpec((R, D3), lambda i: (i, 0)),
        out_shape=jax.ShapeDtypeStruct((N, D3), jnp.float32),
    )(pa, pb, w2, b2, c2, c2b, w3)


def _tc_l3_body(p_ref, b3_ref, c_ref, cb_ref, ib_ref, out_ref,
                sums_ref, cnts_ref):
    i = pl.program_id(0)

    @pl.when(i == 0)
    def _():
        sums_ref[...] = jnp.zeros((G, D3), jnp.float32)
        cnts_ref[...] = jnp.zeros((G, D3), jnp.float32)

    a = p_ref[0] + p_ref[1]
    t = jnp.maximum(a + b3_ref[...], 0.0)
    cm = _conv_mat(c_ref, D3)
    y = jnp.dot(t, cm, preferred_element_type=jnp.float32) + cb_ref[0, 0]
    h = jnp.maximum(y, 0.0)
    ids = ib_ref[0]
    gids = lax.broadcasted_iota(jnp.int32, (G, R), 0)
    onehot = jnp.where(gids == ids, 1.0, 0.0)
    sums_ref[...] = sums_ref[...] + jnp.dot(
        onehot, h, preferred_element_type=jnp.float32)
    cnts_ref[...] = cnts_ref[...] + jnp.sum(onehot, axis=1, keepdims=True)

    @pl.when(i == N // R - 1)
    def _():
        out_ref[...] = sums_ref[...] / jnp.maximum(cnts_ref[...], 1.0)


def _tc_l3(p, b3, c3, c3b, ib):
    return pl.pallas_call(
        _tc_l3_body,
        grid=(N // R,),
        in_specs=[
            pl.BlockSpec((NC, R, D3), lambda i: (0, i, 0)),
            pl.BlockSpec((1, D3), lambda i: (0, 0)),
            pl.BlockSpec(memory_space=pltpu.SMEM),
            pl.BlockSpec(memory_space=pltpu.SMEM),
            pl.BlockSpec((1, 1, R), lambda i: (i, 0, 0)),
        ],
        out_specs=pl.BlockSpec((G, D3), lambda i: (0, 0)),
        out_shape=jax.ShapeDtypeStruct((G, D3), jnp.float32),
        scratch_shapes=[
            pltpu.VMEM((G, D3), jnp.float32),
            pltpu.VMEM((G, D3), jnp.float32),
        ],
    )(p, b3, c3, c3b, ib)


def kernel(x, edge_index, edge_attr, info_batch, W1, b1, c1_w, c1_b,
           W2, b2, c2_w, c2_b, W3, b3, c3_w, c3_b):
    loop = jnp.arange(N, dtype=edge_index.dtype)
    pad = EP - (E_RAW + N)
    src = jnp.concatenate([edge_index[0], loop,
                           jnp.zeros((pad,), edge_index.dtype)])
    dst = jnp.concatenate([edge_index[1], loop,
                           jnp.zeros((pad,), edge_index.dtype)])
    ew = jnp.concatenate([edge_attr, jnp.ones((N,), edge_attr.dtype),
                          jnp.zeros((pad,), edge_attr.dtype)])

    degp = _sc_deg(dst, ew).reshape(NC * NS, N)
    dis = _tc_dis(degp).reshape(N)
    norm = _sc_norm(src, dst, ew, dis)

    aggx = _sc_agg(x, src, dst, norm).reshape(NC, N, D_IN)
    h1a, h1b = _tc_l1(aggx, W1, b1.reshape(1, D1),
                      c1_w.reshape(1, 3), c1_b.reshape(1, 1))

    agga = _sc_agg(h1a, src, dst, norm).reshape(NC, N, D_IN)
    aggb = _sc_agg(h1b, src, dst, norm).reshape(NC, N, D_IN)
    g3 = _tc_l2(agga, aggb, W2, b2.reshape(1, D2),
                c2_w.reshape(1, 3), c2_b.reshape(1, 1), W3)

    aggg = _sc_agg(g3, src, dst, norm).reshape(NC, N, D3)
    ib = info_batch.reshape(N // R, 1, R)
    return _tc_l3(aggg, b3.reshape(1, D3), c3_w.reshape(1, 3),
                  c3_b.reshape(1, 1), ib)
